# baseline (speedup 1.0000x reference)
import dataclasses

import jax
import jax.numpy as jnp
from jax import lax
from jax.experimental import pallas as pl
from jax.experimental.pallas import tpu as pltpu
from jax.experimental.pallas import tpu_sc as plsc

BATCH = 16384
F = 16
PACK = 8
LANES = 128
NC = 2
NS = 16
NW = NC * NS
ROWS_PER_W = BATCH // NW
IDX_CHUNK = 128
N_CHUNKS = ROWS_PER_W // IDX_CHUNK
GROUPS = IDX_CHUNK // F


def _gmf_sc(uhi, uoff, ihi, ioff, ue2d, ie2d, params):
    mesh = plsc.VectorSubcoreMesh(core_axis_name="c", subcore_axis_name="s")
    cp = pltpu.CompilerParams()
    if "needs_layout_passes" in pltpu.CompilerParams.__dataclass_fields__:
        cp = dataclasses.replace(cp, needs_layout_passes=False)

    @pl.kernel(
        compiler_params=cp,
        out_type=jax.ShapeDtypeStruct((BATCH,), jnp.float32),
        mesh=mesh,
        scratch_types=[
            pltpu.VMEM((N_CHUNKS, IDX_CHUNK), jnp.int32),
            pltpu.VMEM((N_CHUNKS, IDX_CHUNK), jnp.int32),
            pltpu.VMEM((N_CHUNKS, IDX_CHUNK), jnp.int32),
            pltpu.VMEM((N_CHUNKS, IDX_CHUNK), jnp.int32),
            pltpu.VMEM((2, IDX_CHUNK, LANES), jnp.float32),
            pltpu.VMEM((2, IDX_CHUNK, LANES), jnp.float32),
            pltpu.VMEM((ROWS_PER_W,), jnp.float32),
            pltpu.VMEM((F + 1, F), jnp.float32),
            pltpu.SemaphoreType.DMA,
            pltpu.SemaphoreType.DMA,
        ],
    )
    def k(uhi_hbm, uoff_hbm, ihi_hbm, ioff_hbm, ue_hbm, ie_hbm, par_hbm,
          out_hbm, uhi_v, uoff_v, ihi_v, ioff_v, u_raw, i_raw, out_v, par_v,
          sem0, sem1):
        wid = lax.axis_index("s") * NC + lax.axis_index("c")
        row0 = wid * N_CHUNKS
        pltpu.sync_copy(uhi_hbm.at[pl.ds(row0, N_CHUNKS), :], uhi_v)
        pltpu.sync_copy(uoff_hbm.at[pl.ds(row0, N_CHUNKS), :], uoff_v)
        pltpu.sync_copy(ihi_hbm.at[pl.ds(row0, N_CHUNKS), :], ihi_v)
        pltpu.sync_copy(ioff_hbm.at[pl.ds(row0, N_CHUNKS), :], ioff_v)
        pltpu.sync_copy(par_hbm, par_v)

        sems = (sem0, sem1)

        def fire(chunk, buf):
            cu = pltpu.async_copy(ue_hbm.at[uhi_v.at[chunk]], u_raw.at[buf],
                                  sems[buf])
            ci = pltpu.async_copy(ie_hbm.at[ihi_v.at[chunk]], i_raw.at[buf],
                                  sems[buf])
            return (cu, ci)

        bvec = par_v[F]
        base_rows = lax.iota(jnp.int32, F)

        pending = [fire(0, 0), fire(1, 1)]

        for chunk in range(N_CHUNKS):
            buf = chunk % 2
            for cpy in pending[chunk]:
                cpy.wait()
            ub = u_raw.at[buf]
            ib = i_raw.at[buf]
            for g in range(GROUPS):
                rows = base_rows + (g * F)
                uo = uoff_v[chunk, pl.ds(g * F, F)]
                io = ioff_v[chunk, pl.ds(g * F, F)]
                acc = bvec
                for f in range(F):
                    ucol = plsc.load_gather(ub, [rows, uo + f])
                    icol = plsc.load_gather(ib, [rows, io + f])
                    acc = acc + ucol * icol * par_v[f]
                out_v[pl.ds(chunk * IDX_CHUNK + g * F, F)] = acc
            if chunk + 2 < N_CHUNKS:
                pending.append(fire(chunk + 2, buf))

        pltpu.sync_copy(out_v, out_hbm.at[pl.ds(wid * ROWS_PER_W, ROWS_PER_W)])

    return k(uhi, uoff, ihi, ioff, ue2d, ie2d, params)


@jax.jit
def kernel(user, item, user_emb, item_emb, W, b):
    user = user.astype(jnp.int32)
    item = item.astype(jnp.int32)
    uhi = (user // PACK).reshape(NW * N_CHUNKS, IDX_CHUNK)
    uoff = ((user % PACK) * F).reshape(NW * N_CHUNKS, IDX_CHUNK)
    ihi = (item // PACK).reshape(NW * N_CHUNKS, IDX_CHUNK)
    ioff = ((item % PACK) * F).reshape(NW * N_CHUNKS, IDX_CHUNK)
    ue2d = user_emb.reshape(user_emb.shape[0] // PACK, LANES)
    ie2d = item_emb.reshape(item_emb.shape[0] // PACK, LANES)
    wcol = W.reshape(F, 1)
    params = jnp.concatenate(
        [jnp.broadcast_to(wcol, (F, F)),
         jnp.broadcast_to(b.reshape(1, 1), (1, F))], axis=0)
    return _gmf_sc(uhi, uoff, ihi, ioff, ue2d, ie2d, params)


# speedup vs baseline: 4.2015x; 4.2015x over previous
import dataclasses

import jax
import jax.numpy as jnp
from jax import lax
from jax.experimental import pallas as pl
from jax.experimental.pallas import tpu as pltpu
from jax.experimental.pallas import tpu_sc as plsc

BATCH = 16384
F = 16
LANES = 128
NC = 2
NS = 16
NW = NC * NS
RPW = BATCH // NW
GRP = 8
PAIRS = RPW // (2 * GRP)

_DNUMS = lax.GatherDimensionNumbers(
    offset_dims=(), collapsed_slice_dims=(0,), start_index_map=(0,))


def _bcast_lane(v, e):
    idx = jnp.full((F, 1), e, jnp.int32)
    return lax.gather(v, idx, dimension_numbers=_DNUMS, slice_sizes=(1,),
                      mode=lax.GatherScatterMode.PROMISE_IN_BOUNDS)


def _gmf_sc(user2d, item2d, ue_t, ie_t, params, dummy):
    mesh = plsc.VectorSubcoreMesh(core_axis_name="c", subcore_axis_name="s")
    cp = pltpu.CompilerParams()
    if "needs_layout_passes" in pltpu.CompilerParams.__dataclass_fields__:
        cp = dataclasses.replace(cp, needs_layout_passes=False)

    @pl.kernel(
        compiler_params=cp,
        out_type=jax.ShapeDtypeStruct((BATCH,), jnp.float32),
        mesh=mesh,
        scratch_types=[
            pltpu.VMEM((RPW,), jnp.int32),
            pltpu.VMEM((RPW,), jnp.int32),
            pltpu.VMEM((F,), jnp.int32),
            pltpu.VMEM((GRP, F, LANES), jnp.float32),
            pltpu.VMEM((GRP, F, LANES), jnp.float32),
            pltpu.VMEM((GRP, F, LANES), jnp.float32),
            pltpu.VMEM((GRP, F, LANES), jnp.float32),
            pltpu.VMEM((F,), jnp.float32),
            pltpu.VMEM((RPW,), jnp.float32),
            pltpu.VMEM((2, F), jnp.float32),
            pltpu.SemaphoreType.DMA,
            pltpu.SemaphoreType.DMA,
        ],
    )
    def k(user_hbm, item_hbm, ue_hbm, ie_hbm, par_hbm, dummy_hbm, out_hbm,
          u_idx, i_idx, fidx_v, ublk0, ublk1, iblk0, iblk1, accv, out_v,
          par_v, sem0, sem1):
        wid = lax.axis_index("s") * NC + lax.axis_index("c")
        pltpu.sync_copy(user_hbm.at[wid], u_idx)
        pltpu.sync_copy(item_hbm.at[wid], i_idx)
        pltpu.sync_copy(par_hbm, par_v)
        lanes = lax.iota(jnp.int32, F)
        fidx_v[...] = lanes

        ubufs = (ublk0, ublk1)
        ibufs = (iblk0, iblk1)
        sems = (sem0, sem1)

        def fire(pair, grp, buf):
            base = pl.multiple_of(pair * 2 * GRP, 2 * GRP)
            ub = u_idx[pl.ds(base, F)] & ~(LANES - 1)
            ib = i_idx[pl.ds(base, F)] & ~(LANES - 1)
            for e in range(grp * GRP, (grp + 1) * GRP):
                mask = lanes == e
                bu = pl.multiple_of(
                    jnp.sum(jnp.where(mask, ub, 0)), LANES)
                bi = pl.multiple_of(
                    jnp.sum(jnp.where(mask, ib, 0)), LANES)
                pltpu.async_copy(ue_hbm.at[fidx_v, pl.ds(bu, LANES)],
                                 ubufs[buf].at[e - grp * GRP], sems[buf])
                pltpu.async_copy(ie_hbm.at[fidx_v, pl.ds(bi, LANES)],
                                 ibufs[buf].at[e - grp * GRP], sems[buf])

        def drain(buf):
            pltpu.make_async_copy(dummy_hbm, ubufs[buf], sems[buf]).wait()
            pltpu.make_async_copy(dummy_hbm, ibufs[buf], sems[buf]).wait()

        wvec = par_v[0]
        bvec = par_v[1]
        accv[...] = bvec

        def compute(pair, grp, buf):
            base = pl.multiple_of(pair * 2 * GRP, 2 * GRP)
            ulu = u_idx[pl.ds(base, F)] & (LANES - 1)
            uli = i_idx[pl.ds(base, F)] & (LANES - 1)
            a = accv[...]
            for e in range(grp * GRP, (grp + 1) * GRP):
                ucol = plsc.load_gather(
                    ubufs[buf].at[e - grp * GRP], [lanes, _bcast_lane(ulu, e)])
                icol = plsc.load_gather(
                    ibufs[buf].at[e - grp * GRP], [lanes, _bcast_lane(uli, e)])
                s = jnp.sum(ucol * icol * wvec)
                a = a + jnp.where(lanes == e, s, 0.0)
            accv[...] = a

        fire(0, 0, 0)
        fire(0, 1, 1)

        @pl.loop(0, PAIRS)
        def _(kk):
            drain(0)
            compute(kk, 0, 0)

            @pl.when(kk < PAIRS - 1)
            def _():
                fire(kk + 1, 0, 0)

            drain(1)
            compute(kk, 1, 1)

            @pl.when(kk < PAIRS - 1)
            def _():
                fire(kk + 1, 1, 1)

            base = pl.multiple_of(kk * F, F)
            out_v[pl.ds(base, F)] = accv[...]
            accv[...] = bvec

        pltpu.sync_copy(out_v, out_hbm.at[pl.ds(wid * RPW, RPW)])

    return k(user2d, item2d, ue_t, ie_t, params, dummy)


@jax.jit
def kernel(user, item, user_emb, item_emb, W, b):
    user2d = user.astype(jnp.int32).reshape(NW, RPW)
    item2d = item.astype(jnp.int32).reshape(NW, RPW)
    ue_t = user_emb.T
    ie_t = item_emb.T
    params = jnp.concatenate(
        [W.reshape(1, F), jnp.broadcast_to(b.reshape(1, 1), (1, F))], axis=0)
    dummy = jnp.zeros((GRP, F, LANES), jnp.float32)
    return _gmf_sc(user2d, item2d, ue_t, ie_t, params, dummy)
